# baseline (speedup 1.0000x reference)
import functools

import jax
import jax.numpy as jnp
from jax import lax
from jax.experimental import pallas as pl
from jax.experimental.pallas import tpu as pltpu
from jax.experimental.pallas import tpu_sc as plsc

N = 10000
E = 320000

NC = 2
NS = 16
NW = NC * NS
EPW = E // NW
K = 40
NBUF = 4
CH = 252
EPW_PAD = CH * K
E_PAD = NW * EPW_PAD
N_PAD = 10240
ROWS_PER_SUB = N_PAD // NS


@functools.lru_cache(maxsize=None)
def _make_edge_agg(H):
  mesh = plsc.VectorSubcoreMesh(core_axis_name="c", subcore_axis_name="s")

  @functools.partial(
      pl.kernel,
      out_type=jax.ShapeDtypeStruct((NC, N_PAD, H), jnp.float32),
      mesh=mesh,
      scratch_types=[
          pltpu.VMEM((2, CH, K), jnp.int32),
          pltpu.VMEM((NBUF, K, H), jnp.float32),
          pltpu.VMEM_SHARED((N_PAD, H), jnp.float32),
      ] + [pltpu.SemaphoreType.DMA] * (2 * NBUF),
      compiler_params=pltpu.CompilerParams(use_tc_tiling_on_sc=False),
  )
  def edge_agg(h_hbm, idx_hbm, zeros_hbm, out_hbm, idx_v, rows_v, acc_sh, *sems):
    gsem = sems[:NBUF]
    ssem = sems[NBUF:]
    c = lax.axis_index("c")
    s = lax.axis_index("s")
    wid = s * NC + c

    pltpu.sync_copy(idx_hbm.at[wid], idx_v)

    def fire_g(jj, b):
      pltpu.async_copy(h_hbm.at[idx_v.at[0, jj]], rows_v.at[b], gsem[b])

    def wait_g(b):
      pltpu.make_async_copy(h_hbm.at[idx_v.at[0, 0]], rows_v.at[b],
                            gsem[b]).wait()

    def fire_s(b, jj):
      pltpu.async_copy(rows_v.at[b], acc_sh.at[idx_v.at[1, jj]], ssem[b],
                       add=True)

    def wait_s(b):
      pltpu.make_async_copy(rows_v.at[b], acc_sh.at[idx_v.at[1, 0]],
                            ssem[b]).wait()

    fire_g(0, 0)
    fire_g(1, 1)

    pltpu.sync_copy(zeros_hbm.at[pl.ds(s * ROWS_PER_SUB, ROWS_PER_SUB)],
                    acc_sh.at[pl.ds(s * ROWS_PER_SUB, ROWS_PER_SUB)])
    plsc.subcore_barrier()

    fire_g(2, 2)
    wait_g(0)
    fire_s(0, 0)
    fire_g(3, 3)
    wait_g(1)
    fire_s(1, 1)

    def step(g, carry):
      t0 = 2 + 4 * g
      for u in range(4):
        b = (2 + u) % 4
        bn = (b + 2) % 4
        wait_s(bn)
        fire_g(t0 + u + 2, bn)
        wait_g(b)
        fire_s(b, t0 + u)
      return carry

    lax.fori_loop(0, (CH - 4) // 4, step, 0)
    wait_s(0)
    wait_g(2)
    fire_s(2, CH - 2)
    wait_s(1)
    wait_g(3)
    fire_s(3, CH - 1)
    wait_s(2)
    wait_s(3)
    plsc.subcore_barrier()

    pltpu.sync_copy(acc_sh.at[pl.ds(s * ROWS_PER_SUB, ROWS_PER_SUB)],
                    out_hbm.at[c, pl.ds(s * ROWS_PER_SUB, ROWS_PER_SUB)])

  return edge_agg


def _tc_first_body(x_ref, w_ref, norm_ref, out_ref):
  out_ref[...] = jnp.dot(x_ref[...], w_ref[...],
                         preferred_element_type=jnp.float32) * norm_ref[...]


def _tc_mid_body(p_ref, norm_ref, b_ref, w_ref, out_ref):
  x = p_ref[0, :N, :] + p_ref[1, :N, :]
  nrm = norm_ref[...]
  x = x * nrm + b_ref[...][None, :]
  col_mean = jnp.mean(x, axis=0)
  rn = jnp.sqrt(1e-6 + jnp.sum(x * x, axis=1, keepdims=True))
  x = x / rn - col_mean[None, :]
  x = jnp.maximum(x, 0.0)
  out_ref[...] = jnp.dot(x, w_ref[...],
                         preferred_element_type=jnp.float32) * nrm


def _tc_final_body(p_ref, norm_ref, b_ref, out_ref):
  x = p_ref[0, :N, :] + p_ref[1, :N, :]
  out_ref[...] = x * norm_ref[...] + b_ref[...][None, :]


def _tc_call(body, out_shape, *args):
  return pl.pallas_call(
      body,
      out_shape=jax.ShapeDtypeStruct(out_shape, jnp.float32),
  )(*args)


@jax.jit
def kernel(in_feat, edge_index, norm, W1, b1, W2, b2, W3, b3):
  pad = E_PAD - E
  src = jnp.concatenate([edge_index[0], jnp.zeros((pad,), jnp.int32)])
  dst = jnp.concatenate([edge_index[1],
                         jnp.full((pad,), N_PAD - 1, jnp.int32)])
  idx = jnp.stack([src.reshape(NW, CH, K),
                   dst.reshape(NW, CH, K)], axis=1)
  zeros_h = jnp.zeros((N_PAD, 128), jnp.float32)
  zeros_c = jnp.zeros((N_PAD, 64), jnp.float32)

  m = _tc_call(_tc_first_body, (N, 128), in_feat, W1, norm)
  p = _make_edge_agg(128)(m, idx, zeros_h)
  m = _tc_call(_tc_mid_body, (N, 128), p, norm, b1, W2)
  p = _make_edge_agg(128)(m, idx, zeros_h)
  m = _tc_call(_tc_mid_body, (N, 64), p, norm, b2, W3)
  p = _make_edge_agg(64)(m, idx, zeros_c)
  out = _tc_call(_tc_final_body, (N, 64), p, norm, b3)
  return out


# speedup vs baseline: 1.5240x; 1.5240x over previous
import functools

import jax
import jax.numpy as jnp
from jax import lax
from jax.experimental import pallas as pl
from jax.experimental.pallas import tpu as pltpu
from jax.experimental.pallas import tpu_sc as plsc

N = 10000
E = 320000

NC = 2
NS = 16
NW = NC * NS
EPW = E // NW
K = 80
CH = EPW // K
NBUF = 2
N_PAD = 10240
ROWS_PER_SUB = N_PAD // NS


@functools.lru_cache(maxsize=None)
def _make_edge_agg(H):
  mesh = plsc.VectorSubcoreMesh(core_axis_name="c", subcore_axis_name="s")

  @functools.partial(
      pl.kernel,
      out_type=jax.ShapeDtypeStruct((NC, N_PAD, H), jnp.float32),
      mesh=mesh,
      scratch_types=[
          pltpu.VMEM((2, CH, K), jnp.int32),
          pltpu.VMEM((NBUF, K, H), jnp.float32),
          pltpu.VMEM_SHARED((N_PAD, H), jnp.float32),
      ] + [pltpu.SemaphoreType.DMA] * NBUF,
      compiler_params=pltpu.CompilerParams(use_tc_tiling_on_sc=False),
  )
  def edge_agg(h_hbm, idx_hbm, zeros_hbm, out_hbm, idx_v, rows_v, acc_sh, *sems):
    c = lax.axis_index("c")
    s = lax.axis_index("s")
    wid = s * NC + c

    pltpu.sync_copy(idx_hbm.at[wid], idx_v)

    def fire(jj, b):
      pltpu.async_copy(h_hbm.at[idx_v.at[0, jj]], rows_v.at[b], sems[b])

    def wait(b):
      pltpu.make_async_copy(h_hbm.at[idx_v.at[0, 0]], rows_v.at[b], sems[b]).wait()

    for b in range(NBUF):
      fire(b, b)

    pltpu.sync_copy(zeros_hbm.at[pl.ds(s * ROWS_PER_SUB, ROWS_PER_SUB)],
                    acc_sh.at[pl.ds(s * ROWS_PER_SUB, ROWS_PER_SUB)])
    plsc.subcore_barrier()

    def step(i, carry):
      j0 = NBUF * i
      for b in range(NBUF):
        wait(b)
        pltpu.sync_copy(rows_v.at[b], acc_sh.at[idx_v.at[1, j0 + b]], add=True)
        fire((j0 + b + NBUF) % CH, b)
      return carry

    lax.fori_loop(0, CH // NBUF, step, 0)
    j_tail = (CH // NBUF) * NBUF
    for t in range(CH % NBUF):
      b = t
      wait(b)
      pltpu.sync_copy(rows_v.at[b], acc_sh.at[idx_v.at[1, j_tail + t]], add=True)
      fire((j_tail + t + NBUF) % CH, b)
    for b in range(NBUF):
      wait(b)
    plsc.subcore_barrier()

    pltpu.sync_copy(acc_sh.at[pl.ds(s * ROWS_PER_SUB, ROWS_PER_SUB)],
                    out_hbm.at[c, pl.ds(s * ROWS_PER_SUB, ROWS_PER_SUB)])

  return edge_agg


def _tc_first_body(x_ref, w_ref, norm_ref, out_ref):
  out_ref[...] = jnp.dot(x_ref[...], w_ref[...],
                         preferred_element_type=jnp.float32) * norm_ref[...]


def _tc_mid_body(p_ref, norm_ref, b_ref, w_ref, out_ref):
  x = p_ref[0, :N, :] + p_ref[1, :N, :]
  nrm = norm_ref[...]
  x = x * nrm + b_ref[...][None, :]
  col_mean = jnp.mean(x, axis=0)
  rn = jnp.sqrt(1e-6 + jnp.sum(x * x, axis=1, keepdims=True))
  x = x / rn - col_mean[None, :]
  x = jnp.maximum(x, 0.0)
  out_ref[...] = jnp.dot(x, w_ref[...],
                         preferred_element_type=jnp.float32) * nrm


def _tc_final_body(p_ref, norm_ref, b_ref, out_ref):
  x = p_ref[0, :N, :] + p_ref[1, :N, :]
  out_ref[...] = x * norm_ref[...] + b_ref[...][None, :]


def _tc_call(body, out_shape, *args):
  return pl.pallas_call(
      body,
      out_shape=jax.ShapeDtypeStruct(out_shape, jnp.float32),
  )(*args)


@jax.jit
def kernel(in_feat, edge_index, norm, W1, b1, W2, b2, W3, b3):
  src = edge_index[0].reshape(NW, 1, CH, K)
  dst = edge_index[1].reshape(NW, 1, CH, K)
  idx = jnp.concatenate([src, dst], axis=1)
  zeros_h = jnp.zeros((N_PAD, 128), jnp.float32)
  zeros_c = jnp.zeros((N_PAD, 64), jnp.float32)

  m = _tc_call(_tc_first_body, (N, 128), in_feat, W1, norm)
  p = _make_edge_agg(128)(m, idx, zeros_h)
  m = _tc_call(_tc_mid_body, (N, 128), p, norm, b1, W2)
  p = _make_edge_agg(128)(m, idx, zeros_h)
  m = _tc_call(_tc_mid_body, (N, 64), p, norm, b2, W3)
  p = _make_edge_agg(64)(m, idx, zeros_c)
  out = _tc_call(_tc_final_body, (N, 64), p, norm, b3)
  return out


# speedup vs baseline: 1.6722x; 1.0973x over previous
import functools

import jax
import jax.numpy as jnp
from jax import lax
from jax.experimental import pallas as pl
from jax.experimental.pallas import tpu as pltpu
from jax.experimental.pallas import tpu_sc as plsc

N = 10000
E = 320000

NC = 2
NS = 16
NW = NC * NS
EPW = E // NW
K = 40
CH = EPW // K
NBUF = 4
N_PAD = 10240
ROWS_PER_SUB = N_PAD // NS


@functools.lru_cache(maxsize=None)
def _make_edge_agg(H):
  mesh = plsc.VectorSubcoreMesh(core_axis_name="c", subcore_axis_name="s")

  @functools.partial(
      pl.kernel,
      out_type=jax.ShapeDtypeStruct((NC, N_PAD, H), jnp.float32),
      mesh=mesh,
      scratch_types=[
          pltpu.VMEM((2, CH, K), jnp.int32),
          pltpu.VMEM((NBUF, K, H), jnp.float32),
          pltpu.VMEM_SHARED((N_PAD, H), jnp.float32),
      ] + [pltpu.SemaphoreType.DMA] * NBUF,
      compiler_params=pltpu.CompilerParams(use_tc_tiling_on_sc=False),
  )
  def edge_agg(h_hbm, idx_hbm, zeros_hbm, out_hbm, idx_v, rows_v, acc_sh, *sems):
    c = lax.axis_index("c")
    s = lax.axis_index("s")
    wid = s * NC + c

    pltpu.sync_copy(idx_hbm.at[wid], idx_v)

    def fire(jj, b):
      pltpu.async_copy(h_hbm.at[idx_v.at[0, jj]], rows_v.at[b], sems[b])

    def wait(b):
      pltpu.make_async_copy(h_hbm.at[idx_v.at[0, 0]], rows_v.at[b], sems[b]).wait()

    for b in range(NBUF):
      fire(b, b)

    pltpu.sync_copy(zeros_hbm.at[pl.ds(s * ROWS_PER_SUB, ROWS_PER_SUB)],
                    acc_sh.at[pl.ds(s * ROWS_PER_SUB, ROWS_PER_SUB)])
    plsc.subcore_barrier()

    def step(i, carry):
      j0 = NBUF * i
      for b in range(NBUF):
        wait(b)
        pltpu.sync_copy(rows_v.at[b], acc_sh.at[idx_v.at[1, j0 + b]], add=True)
        fire((j0 + b + NBUF) % CH, b)
      return carry

    lax.fori_loop(0, CH // NBUF, step, 0)
    j_tail = (CH // NBUF) * NBUF
    for t in range(CH % NBUF):
      b = t
      wait(b)
      pltpu.sync_copy(rows_v.at[b], acc_sh.at[idx_v.at[1, j_tail + t]], add=True)
      fire((j_tail + t + NBUF) % CH, b)
    for b in range(NBUF):
      wait(b)
    plsc.subcore_barrier()

    pltpu.sync_copy(acc_sh.at[pl.ds(s * ROWS_PER_SUB, ROWS_PER_SUB)],
                    out_hbm.at[c, pl.ds(s * ROWS_PER_SUB, ROWS_PER_SUB)])

  return edge_agg


def _tc_first_body(x_ref, w_ref, norm_ref, out_ref):
  out_ref[...] = jnp.dot(x_ref[...], w_ref[...],
                         preferred_element_type=jnp.float32) * norm_ref[...]


def _tc_mid_body(p_ref, norm_ref, b_ref, w_ref, out_ref):
  x = p_ref[0, :N, :] + p_ref[1, :N, :]
  nrm = norm_ref[...]
  x = x * nrm + b_ref[...][None, :]
  col_mean = jnp.mean(x, axis=0)
  rn = jnp.sqrt(1e-6 + jnp.sum(x * x, axis=1, keepdims=True))
  x = x / rn - col_mean[None, :]
  x = jnp.maximum(x, 0.0)
  out_ref[...] = jnp.dot(x, w_ref[...],
                         preferred_element_type=jnp.float32) * nrm


def _tc_final_body(p_ref, norm_ref, b_ref, out_ref):
  x = p_ref[0, :N, :] + p_ref[1, :N, :]
  out_ref[...] = x * norm_ref[...] + b_ref[...][None, :]


def _tc_call(body, out_shape, *args):
  return pl.pallas_call(
      body,
      out_shape=jax.ShapeDtypeStruct(out_shape, jnp.float32),
  )(*args)


@jax.jit
def kernel(in_feat, edge_index, norm, W1, b1, W2, b2, W3, b3):
  src = edge_index[0].reshape(NW, 1, CH, K)
  dst = edge_index[1].reshape(NW, 1, CH, K)
  idx = jnp.concatenate([src, dst], axis=1)
  zeros_h = jnp.zeros((N_PAD, 128), jnp.float32)
  zeros_c = jnp.zeros((N_PAD, 64), jnp.float32)

  m = _tc_call(_tc_first_body, (N, 128), in_feat, W1, norm)
  p = _make_edge_agg(128)(m, idx, zeros_h)
  m = _tc_call(_tc_mid_body, (N, 128), p, norm, b1, W2)
  p = _make_edge_agg(128)(m, idx, zeros_h)
  m = _tc_call(_tc_mid_body, (N, 64), p, norm, b2, W3)
  p = _make_edge_agg(64)(m, idx, zeros_c)
  out = _tc_call(_tc_final_body, (N, 64), p, norm, b3)
  return out
